# baseline (speedup 1.0000x reference)
import functools

import jax
import jax.numpy as jnp
from jax import lax
from jax.experimental import pallas as pl
from jax.experimental.pallas import tpu as pltpu
from jax.experimental.pallas import tpu_sc as plsc

_NC = 2
_NS = 16
_NW = _NC * _NS
_LANES = 16

_VOCAB = 1000000
_D = 64
_SEQ = 200
_BATCH = 16384
_NCLS = 1000

_BAGS_PER_W = _BATCH // _NW
_CB = 2
_CHUNK_ROWS = _CB * _SEQ
_CHUNKS = _BAGS_PER_W // _CB
_G0 = 128
_G1 = _SEQ - _G0


_TCOLS = _VOCAB // 128
_VTAIL = _VOCAB - _TCOLS * 128
_NCOLS = _TCOLS + 1
_COLS_BASE = _NCOLS // _NW
_COLS_REM = _NCOLS - _COLS_BASE * _NW


def _make_sc_deband():
  mesh = plsc.VectorSubcoreMesh(
      core_axis_name="c", subcore_axis_name="s",
      num_cores=_NC, num_subcores=_NS)

  @functools.partial(
      pl.kernel,
      out_type=jax.ShapeDtypeStruct((_NCOLS, _D, 128), jnp.float32),
      mesh=mesh,
      compiler_params=pltpu.CompilerParams(use_tc_tiling_on_sc=True),
      scratch_types=[pltpu.SemaphoreType.DMA],
  )
  def sc_deband(tt_hbm, tail_hbm, out_hbm, sem):
    cid = lax.axis_index("c")
    sid = lax.axis_index("s")
    wid = sid * _NC + cid
    start = wid * _COLS_BASE + jnp.minimum(wid, _COLS_REM)
    n = _COLS_BASE + jnp.where(wid < _COLS_REM, 1, 0)

    def body(c, carry):
      gcol = start + c
      @pl.when(gcol < _TCOLS)
      def _():
        pltpu.make_async_copy(
            tt_hbm.at[pl.ds(0, _D), pl.ds(gcol * 128, 128)],
            out_hbm.at[gcol], sem).start()
      @pl.when(gcol == _TCOLS)
      def _():
        pltpu.make_async_copy(tail_hbm, out_hbm.at[gcol], sem).start()
      return carry
    lax.fori_loop(0, n, body, 0)

    def drain(c, carry):
      pltpu.make_async_copy(
          out_hbm.at[start + c], out_hbm.at[start + c], sem).wait()
      return carry
    lax.fori_loop(0, n, drain, 0)

  return sc_deband


def _make_sc_transpose():
  mesh = plsc.VectorSubcoreMesh(
      core_axis_name="c", subcore_axis_name="s",
      num_cores=_NC, num_subcores=_NS)

  @functools.partial(
      pl.kernel,
      out_type=jax.ShapeDtypeStruct((_NCOLS * 128 * _D,), jnp.float32),
      mesh=mesh,
      compiler_params=pltpu.CompilerParams(
          use_tc_tiling_on_sc=False, needs_layout_passes=False),
      scratch_types=[
          pltpu.VMEM((2, _D * 128), jnp.float32),
          pltpu.VMEM((2, 128 * _D), jnp.float32),
          pltpu.SemaphoreType.DMA,
          pltpu.SemaphoreType.DMA,
          pltpu.SemaphoreType.DMA,
      ],
  )
  def sc_transpose(slab_hbm, out_hbm, in_v, out_vv, isem, osem0, osem1):
    cid = lax.axis_index("c")
    sid = lax.axis_index("s")
    wid = sid * _NC + cid
    start = wid * _COLS_BASE + jnp.minimum(wid, _COLS_REM)
    n = _COLS_BASE + jnp.where(wid < _COLS_REM, 1, 0)
    osems = [osem0, osem1]
    lane128 = lax.iota(jnp.int32, 16) * 128
    nw = _D * 128

    def in_start(c, s):
      pltpu.make_async_copy(
          slab_hbm.at[pl.ds((start + c) * nw, nw)], in_v.at[s], isem).start()

    def in_wait(s):
      pltpu.make_async_copy(
          slab_hbm.at[pl.ds(0, nw)], in_v.at[s], isem).wait()

    def out_start(c, s):
      pltpu.make_async_copy(
          out_vv.at[s], out_hbm.at[pl.ds((start + c) * nw, nw)],
          osems[s]).start()

    def out_wait(s):
      pltpu.make_async_copy(
          out_vv.at[s], out_hbm.at[pl.ds(0, nw)], osems[s]).wait()

    def transpose(s):
      def vbody(v, carry):
        for q in range(_D // 16):
          idx = lane128 + (q * 16 * 128 + v)
          vals = plsc.load_gather(in_v.at[s], [idx])
          out_vv[s, pl.ds(v * _D + q * 16, 16)] = vals
        return carry
      lax.fori_loop(0, 128, vbody, 0)

    def step(c, s):
      @pl.when(c + 1 < n)
      def _():
        in_start(c + 1, 1 - s)
      in_wait(s)
      @pl.when(c >= 2)
      def _():
        out_wait(s)
      transpose(s)
      out_start(c, s)

    in_start(0, 0)
    def loop(k, carry):
      step(2 * k, 0)
      step(2 * k + 1, 1)
      return carry
    lax.fori_loop(0, n // 2, loop, 0)
    @pl.when(n % 2 == 1)
    def _():
      step(n - 1, 0)
    out_wait(0)
    out_wait(1)

  return sc_transpose


def _make_sc_bag_sum():
  mesh = plsc.VectorSubcoreMesh(
      core_axis_name="c", subcore_axis_name="s",
      num_cores=_NC, num_subcores=_NS)

  @functools.partial(
      pl.kernel,
      out_type=jax.ShapeDtypeStruct((_BATCH, _D), jnp.float32),
      mesh=mesh,
      compiler_params=pltpu.CompilerParams(use_tc_tiling_on_sc=False),
      scratch_types=[
          pltpu.VMEM((2, _CB * 128), jnp.int32),
          pltpu.VMEM((2, _CB * 128), jnp.int32),
          pltpu.VMEM((2, _CHUNK_ROWS, _D), jnp.float32),
          pltpu.VMEM((_BAGS_PER_W, _D), jnp.float32),
          pltpu.SemaphoreType.DMA,
          pltpu.SemaphoreType.DMA,
          pltpu.SemaphoreType.DMA,
      ],
  )
  def sc_bag_sum(texta_hbm, textb_hbm, table_hbm, out_hbm,
                 idxa_v, idxb_v, rows_v, out_v, isem, gsem0, gsem1):
    cid = lax.axis_index("c")
    sid = lax.axis_index("s")
    wid = sid * _NC + cid
    bag0 = wid * _BAGS_PER_W
    gsems = [gsem0, gsem1]

    def idx_copy_start(c, s):
      off = (bag0 + c * _CB) * 128
      pltpu.make_async_copy(
          texta_hbm.at[pl.ds(off, _CB * 128)], idxa_v.at[s], isem).start()
      pltpu.make_async_copy(
          textb_hbm.at[pl.ds(off, _CB * 128)], idxb_v.at[s], isem).start()

    def idx_copy_wait(c, s):
      off = (bag0 + c * _CB) * 128
      pltpu.make_async_copy(
          texta_hbm.at[pl.ds(off, _CB * 128)], idxa_v.at[s], isem).wait()
      pltpu.make_async_copy(
          textb_hbm.at[pl.ds(off, _CB * 128)], idxb_v.at[s], isem).wait()

    def gather_start(s):
      for j in range(_CB):
        pltpu.make_async_copy(
            table_hbm.at[idxa_v.at[s, pl.ds(j * 128, _G0)]],
            rows_v.at[s, pl.ds(j * _SEQ, _G0)],
            gsems[s]).start()
        pltpu.make_async_copy(
            table_hbm.at[idxb_v.at[s, pl.ds(j * 128, _G1)]],
            rows_v.at[s, pl.ds(j * _SEQ + _G0, _G1)],
            gsems[s]).start()

    def gather_wait(s):
      pltpu.make_async_copy(
          table_hbm.at[pl.ds(0, _CHUNK_ROWS)], rows_v.at[s], gsems[s]).wait()

    def accumulate(c, s):
      for j in range(_CB):
        def gbody(g, accs):
          base = j * _SEQ + g * 8
          out = []
          for q in range(_D // _LANES):
            sl = pl.ds(q * _LANES, _LANES)
            v = [rows_v[s, base + i, sl] for i in range(8)]
            gsum = ((v[0] + v[1]) + (v[2] + v[3])) + ((v[4] + v[5]) + (v[6] + v[7]))
            out.append(accs[q] + gsum)
          return tuple(out)
        zero = jnp.zeros((_LANES,), jnp.float32)
        accs = lax.fori_loop(0, _SEQ // 8, gbody, (zero,) * (_D // _LANES))
        for q in range(_D // _LANES):
          out_v[c * _CB + j, pl.ds(q * _LANES, _LANES)] = accs[q]

    def step(c, s):
      @pl.when(c + 1 < _CHUNKS)
      def _():
        idx_copy_wait(c + 1, 1 - s)
        gather_start(1 - s)
      gather_wait(s)
      @pl.when(c + 2 < _CHUNKS)
      def _():
        idx_copy_start(c + 2, s)
      accumulate(c, s)

    idx_copy_start(0, 0)
    idx_copy_wait(0, 0)
    gather_start(0)
    idx_copy_start(1, 1)

    def loop(k, carry):
      step(2 * k, 0)
      step(2 * k + 1, 1)
      return carry
    lax.fori_loop(0, _CHUNKS // 2, loop, 0)

    pltpu.sync_copy(out_v, out_hbm.at[pl.ds(bag0, _BAGS_PER_W)])

  return sc_bag_sum


def _tc_head(text, sums, Wt, b2d):
  BB = 512
  grid = (_BATCH // BB,)

  def body(text_ref, sums_ref, w_ref, b_ref, out_ref):
    t = text_ref[...]
    cnt = jnp.sum((t != 0).astype(jnp.float32), axis=1, keepdims=True)
    mean = sums_ref[...] * (1.0 / jnp.maximum(cnt, 1.0))
    out_ref[...] = lax.dot_general(
        mean, w_ref[...], (((1,), (0,)), ((), ())),
        preferred_element_type=jnp.float32) + b_ref[...]

  return pl.pallas_call(
      body,
      grid=grid,
      in_specs=[
          pl.BlockSpec((BB, _SEQ), lambda i: (i, 0)),
          pl.BlockSpec((BB, _D), lambda i: (i, 0)),
          pl.BlockSpec((_D, _NCLS), lambda i: (0, 0)),
          pl.BlockSpec((1, _NCLS), lambda i: (0, 0)),
      ],
      out_specs=pl.BlockSpec((BB, _NCLS), lambda i: (i, 0)),
      out_shape=jax.ShapeDtypeStruct((_BATCH, _NCLS), jnp.float32),
  )(text, sums, Wt, b2d)


_sc_deband = _make_sc_deband()
_sc_transpose = _make_sc_transpose()
_sc_bag_sum = _make_sc_bag_sum()


def kernel(text, table, W, b):
  text = text.astype(jnp.int32)
  texta = lax.slice(text, (0, 0), (_BATCH, _G0)).reshape(-1)
  textb = jnp.pad(lax.slice(text, (0, _G0), (_BATCH, _SEQ)),
                  ((0, 0), (0, 128 - _G1))).reshape(-1)
  table_t = table.T
  tail128 = jnp.pad(
      lax.slice(table_t, (0, _TCOLS * 128), (_D, _VOCAB)),
      ((0, 0), (0, 128 - _VTAIL)))
  slabs = _sc_deband(table_t, tail128)
  table_lin = _sc_transpose(slabs.reshape(-1))
  sums = _sc_bag_sum(texta, textb, table_lin.reshape(_NCOLS * 128, _D))
  return _tc_head(text, sums, W.T, b.reshape(1, _NCLS))


# speedup vs baseline: 9.4549x; 9.4549x over previous
import functools

import jax
import jax.numpy as jnp
from jax import lax
from jax.experimental import pallas as pl
from jax.experimental.pallas import tpu as pltpu
from jax.experimental.pallas import tpu_sc as plsc

_NC = 2
_NS = 16
_NW = _NC * _NS
_LANES = 16

_VOCAB = 1000000
_D = 64
_SEQ = 200
_BATCH = 16384
_NCLS = 1000

_BAGS_PER_W = _BATCH // _NW
_CB = 2
_CHUNK_ROWS = _CB * _SEQ
_CHUNKS = _BAGS_PER_W // _CB
_G0 = 128
_G1 = _SEQ - _G0


_TCOLS = _VOCAB // 128
_VTAIL = _VOCAB - _TCOLS * 128
_NCOLS = _TCOLS + 1
_COLS_BASE = _NCOLS // _NW
_COLS_REM = _NCOLS - _COLS_BASE * _NW


def _make_sc_bag_sum():
  mesh = plsc.VectorSubcoreMesh(
      core_axis_name="c", subcore_axis_name="s",
      num_cores=_NC, num_subcores=_NS)

  @functools.partial(
      pl.kernel,
      out_type=jax.ShapeDtypeStruct((_BATCH, _D), jnp.float32),
      mesh=mesh,
      compiler_params=pltpu.CompilerParams(use_tc_tiling_on_sc=False),
      scratch_types=[
          pltpu.VMEM((2, _CB * 128), jnp.int32),
          pltpu.VMEM((2, _CB * 128), jnp.int32),
          pltpu.VMEM((2, _CHUNK_ROWS, _D), jnp.float32),
          pltpu.VMEM((_BAGS_PER_W, _D), jnp.float32),
          pltpu.SemaphoreType.DMA,
          pltpu.SemaphoreType.DMA,
          pltpu.SemaphoreType.DMA,
      ],
  )
  def sc_bag_sum(texta_hbm, textb_hbm, table_hbm, out_hbm,
                 idxa_v, idxb_v, rows_v, out_v, isem, gsem0, gsem1):
    cid = lax.axis_index("c")
    sid = lax.axis_index("s")
    wid = sid * _NC + cid
    bag0 = wid * _BAGS_PER_W
    gsems = [gsem0, gsem1]

    def idx_copy_start(c, s):
      off = (bag0 + c * _CB) * 128
      pltpu.make_async_copy(
          texta_hbm.at[pl.ds(off, _CB * 128)], idxa_v.at[s], isem).start()
      pltpu.make_async_copy(
          textb_hbm.at[pl.ds(off, _CB * 128)], idxb_v.at[s], isem).start()

    def idx_copy_wait(c, s):
      off = (bag0 + c * _CB) * 128
      pltpu.make_async_copy(
          texta_hbm.at[pl.ds(off, _CB * 128)], idxa_v.at[s], isem).wait()
      pltpu.make_async_copy(
          textb_hbm.at[pl.ds(off, _CB * 128)], idxb_v.at[s], isem).wait()

    def gather_start(s):
      for j in range(_CB):
        pltpu.make_async_copy(
            table_hbm.at[idxa_v.at[s, pl.ds(j * 128, _G0)]],
            rows_v.at[s, pl.ds(j * _SEQ, _G0)],
            gsems[s]).start()
        pltpu.make_async_copy(
            table_hbm.at[idxb_v.at[s, pl.ds(j * 128, _G1)]],
            rows_v.at[s, pl.ds(j * _SEQ + _G0, _G1)],
            gsems[s]).start()

    def gather_wait(s):
      pltpu.make_async_copy(
          table_hbm.at[pl.ds(0, _CHUNK_ROWS)], rows_v.at[s], gsems[s]).wait()

    def accumulate(c, s):
      for j in range(_CB):
        def gbody(g, accs):
          base = j * _SEQ + g * 8
          out = []
          for q in range(_D // _LANES):
            sl = pl.ds(q * _LANES, _LANES)
            v = [rows_v[s, base + i, sl] for i in range(8)]
            gsum = ((v[0] + v[1]) + (v[2] + v[3])) + ((v[4] + v[5]) + (v[6] + v[7]))
            out.append(accs[q] + gsum)
          return tuple(out)
        zero = jnp.zeros((_LANES,), jnp.float32)
        accs = lax.fori_loop(0, _SEQ // 8, gbody, (zero,) * (_D // _LANES))
        for q in range(_D // _LANES):
          out_v[c * _CB + j, pl.ds(q * _LANES, _LANES)] = accs[q]

    def step(c, s):
      @pl.when(c + 1 < _CHUNKS)
      def _():
        idx_copy_wait(c + 1, 1 - s)
        gather_start(1 - s)
      gather_wait(s)
      @pl.when(c + 2 < _CHUNKS)
      def _():
        idx_copy_start(c + 2, s)
      accumulate(c, s)

    idx_copy_start(0, 0)
    idx_copy_wait(0, 0)
    gather_start(0)
    idx_copy_start(1, 1)

    def loop(k, carry):
      step(2 * k, 0)
      step(2 * k + 1, 1)
      return carry
    lax.fori_loop(0, _CHUNKS // 2, loop, 0)

    pltpu.sync_copy(out_v, out_hbm.at[pl.ds(bag0, _BAGS_PER_W)])

  return sc_bag_sum


def _tc_head(texta2, textb2, sums, Wt, bcol):
  BB = 512
  grid = (_BATCH // BB,)

  def body(ta_ref, tb_ref, sums_ref, w_ref, b_ref, out_ref):
    cnt = (jnp.sum((ta_ref[...] != 0).astype(jnp.float32), axis=1,
                   keepdims=True)
           + jnp.sum((tb_ref[...] != 0).astype(jnp.float32), axis=1,
                     keepdims=True))
    mean = sums_ref[...] * (1.0 / jnp.maximum(cnt, 1.0))
    out_ref[...] = lax.dot_general(
        w_ref[...], mean, (((0,), (1,)), ((), ())),
        preferred_element_type=jnp.float32) + b_ref[...]

  return pl.pallas_call(
      body,
      grid=grid,
      in_specs=[
          pl.BlockSpec((BB, 128), lambda i: (i, 0)),
          pl.BlockSpec((BB, 128), lambda i: (i, 0)),
          pl.BlockSpec((BB, _D), lambda i: (i, 0)),
          pl.BlockSpec((_D, _NCLS), lambda i: (0, 0)),
          pl.BlockSpec((_NCLS, 1), lambda i: (0, 0)),
      ],
      out_specs=pl.BlockSpec((_NCLS, BB), lambda i: (0, i)),
      out_shape=jax.ShapeDtypeStruct((_NCLS, _BATCH), jnp.float32),
  )(texta2, textb2, sums, Wt, bcol)


_sc_bag_sum = _make_sc_bag_sum()


def kernel(text, table, W, b):
  text = text.astype(jnp.int32)
  texta = lax.slice(text, (0, 0), (_BATCH, _G0)).reshape(-1)
  textb = jnp.pad(lax.slice(text, (0, _G0), (_BATCH, _SEQ)),
                  ((0, 0), (0, 128 - _G1))).reshape(-1)
  sums = _sc_bag_sum(texta, textb, table)
  out_t = _tc_head(texta.reshape(_BATCH, 128), textb.reshape(_BATCH, 128),
                   sums, W.T, b.reshape(_NCLS, 1))
  return out_t.T
